# baseline (speedup 1.0000x reference)
import jax
import jax.numpy as jnp
from jax.experimental import pallas as pl

_B, _S, _DIN, _E, _G, _C = 16, 4096, 8, 64, 128, 8
_R = _B * _DIN
_CH2 = 512
_CH3 = 256
_PAD = 0.0


def _stats_kernel(x2_ref, lrep_ref, mean_ref, sv_ref):
    f32 = jnp.float32
    x2 = x2_ref[...]
    lrep = lrep_ref[...]
    il = jax.lax.broadcasted_iota(jnp.int32, (_R, _S), 1)
    mf = (il < lrep).astype(f32)
    cnt = jnp.maximum(jnp.sum(mf) * (1.0 / _DIN), 1.0)

    rd = jax.lax.broadcasted_iota(jnp.int32, (_DIN, _R), 1)
    dd = jax.lax.broadcasted_iota(jnp.int32, (_DIN, _R), 0)
    m8 = (jax.lax.rem(rd, _DIN) == dd).astype(f32)
    rr = jax.lax.broadcasted_iota(jnp.int32, (_R, _DIN), 0)
    dc = jax.lax.broadcasted_iota(jnp.int32, (_R, _DIN), 1)
    p8 = (jax.lax.rem(rr, _DIN) == dc).astype(f32)

    hi = jax.lax.Precision.HIGHEST
    sum_rows = jnp.sum(x2 * mf, axis=1, keepdims=True)
    dn = (((1,), (0,)), ((), ()))
    mean8 = jax.lax.dot_general(m8, sum_rows, dn, precision=hi) / cnt
    mean_r = jax.lax.dot_general(p8, mean8, dn, precision=hi)
    xc = x2 - mean_r
    sq_rows = jnp.sum((xc * xc) * mf, axis=1, keepdims=True)
    var8 = jax.lax.dot_general(m8, sq_rows, dn, precision=hi) / cnt
    sv8 = jnp.sqrt(var8 + 1e-6)

    i8r = jax.lax.broadcasted_iota(jnp.int32, (_DIN, _DIN), 0)
    i8c = jax.lax.broadcasted_iota(jnp.int32, (_DIN, _DIN), 1)
    eye8 = (i8r == i8c).astype(f32)
    dt = (((0,), (0,)), ((), ()))
    mean_ref[...] = jax.lax.dot_general(mean8, eye8, dt, precision=hi)
    sv_ref[...] = jax.lax.dot_general(sv8, eye8, dt, precision=hi)


def _mlp_h(xs, mean_row, sv_row, w1, b1r, w2, b2r):
    xn = (xs - mean_row) / sv_row
    h = jnp.maximum(xn @ w1 + b1r, 0.0)
    return jnp.maximum(h @ w2 + b2r, 0.0)


def _gfv_kernel(x_ref, len3_ref, mean_ref, sv_ref, w1_ref, b1r_ref,
                w2_ref, b2r_ref, wg_ref, bgr_ref, gfv_ref):
    k = pl.program_id(0)
    base = k * _CH2
    x = x_ref[...]
    len3 = len3_ref[...]

    n = _B * _CH2
    h = _mlp_h(x.reshape(n, _DIN), mean_ref[...], sv_ref[...],
               w1_ref[...], b1r_ref[...], w2_ref[...], b2r_ref[...])
    g = jnp.maximum(h @ wg_ref[...] + bgr_ref[...], 0.0)

    ig = jax.lax.broadcasted_iota(jnp.int32, (_B, _CH2, _G), 1) + base
    maskg = ig < jnp.broadcast_to(len3, (_B, _CH2, _G))
    g3 = jnp.where(maskg, g.reshape(_B, _CH2, _G), -jnp.inf)
    part = jnp.max(g3, axis=1)

    @pl.when(k == 0)
    def _():
        gfv_ref[...] = jnp.full((_B, _G), -jnp.inf, jnp.float32)

    gfv_ref[...] = jnp.maximum(gfv_ref[...], part)


def _out_kernel(x_ref, len3_ref, mean_ref, sv_ref, w1_ref, b1r_ref,
                w2_ref, b2r_ref, gfv3_ref, wseg_ref, bsegr_ref,
                logits_ref, labels_ref, scores_ref, cat_ref):
    k = pl.program_id(0)
    base = k * _CH3
    x = x_ref[...]
    len3 = len3_ref[...]

    n = _B * _CH3
    h = _mlp_h(x.reshape(n, _DIN), mean_ref[...], sv_ref[...],
               w1_ref[...], b1r_ref[...], w2_ref[...], b2r_ref[...])

    ie = jax.lax.broadcasted_iota(jnp.int32, (_B, _CH3, _E), 1) + base
    maske = ie < jnp.broadcast_to(len3, (_B, _CH3, _E))
    h3 = jnp.where(maske, h.reshape(_B, _CH3, _E), _PAD)

    ig = jax.lax.broadcasted_iota(jnp.int32, (_B, _CH3, _G), 1) + base
    maskg = ig < jnp.broadcast_to(len3, (_B, _CH3, _G))
    gfv3 = jnp.broadcast_to(gfv3_ref[...], (_B, _CH3, _G))
    gfv3 = jnp.where(maskg, gfv3, _PAD)

    cat3 = jnp.concatenate([h3, gfv3], axis=2)
    cat_ref[...] = cat3

    logits = cat3.reshape(n, _E + _G) @ wseg_ref[...] + bsegr_ref[...]
    ic = jax.lax.broadcasted_iota(jnp.int32, (_B, _CH3, _C), 1) + base
    maskc = ic < jnp.broadcast_to(len3, (_B, _CH3, _C))
    logits3 = jnp.where(maskc, logits.reshape(_B, _CH3, _C), _PAD)
    logits_ref[...] = logits3

    m = jnp.max(logits3, axis=2, keepdims=True)
    ssum = jnp.sum(jnp.exp(logits3 - m), axis=2)
    scores = 1.0 / ssum
    scores_ref[...] = scores

    cidx = jax.lax.broadcasted_iota(jnp.int32, (_B, _CH3, _C), 2)
    cand = jnp.where(logits3 == m, cidx, _C)
    labels = jnp.min(cand, axis=2)
    labels_ref[...] = jnp.where(jnp.isnan(scores), -1, labels)


def kernel(x, lengths, W1, b1, W2, b2, Wg, bg, Wseg, bseg):
    f32 = jnp.float32
    x2 = x.transpose(0, 2, 1).reshape(_R, _S)
    lrep = jnp.repeat(lengths.astype(jnp.int32), _DIN).reshape(_R, 1)
    len3 = lengths.astype(jnp.int32).reshape(_B, 1, 1)
    b1r = b1.reshape(1, _E)
    b2r = b2.reshape(1, _E)
    bgr = bg.reshape(1, _G)
    bsegr = bseg.reshape(1, _C)

    mean_row, sv_row = pl.pallas_call(
        _stats_kernel,
        out_shape=[
            jax.ShapeDtypeStruct((1, _DIN), f32),
            jax.ShapeDtypeStruct((1, _DIN), f32),
        ],
    )(x2, lrep)

    n2 = _S // _CH2
    gfv = pl.pallas_call(
        _gfv_kernel,
        grid=(n2,),
        in_specs=[
            pl.BlockSpec((_B, _CH2, _DIN), lambda k: (0, k, 0)),
            pl.BlockSpec((_B, 1, 1), lambda k: (0, 0, 0)),
            pl.BlockSpec((1, _DIN), lambda k: (0, 0)),
            pl.BlockSpec((1, _DIN), lambda k: (0, 0)),
            pl.BlockSpec((_DIN, _E), lambda k: (0, 0)),
            pl.BlockSpec((1, _E), lambda k: (0, 0)),
            pl.BlockSpec((_E, _E), lambda k: (0, 0)),
            pl.BlockSpec((1, _E), lambda k: (0, 0)),
            pl.BlockSpec((_E, _G), lambda k: (0, 0)),
            pl.BlockSpec((1, _G), lambda k: (0, 0)),
        ],
        out_specs=pl.BlockSpec((_B, _G), lambda k: (0, 0)),
        out_shape=jax.ShapeDtypeStruct((_B, _G), f32),
    )(x, len3, mean_row, sv_row, W1, b1r, W2, b2r, Wg, bgr)

    gfv3 = gfv.reshape(_B, 1, _G)

    n3 = _S // _CH3
    logits, labels, scores, cat = pl.pallas_call(
        _out_kernel,
        grid=(n3,),
        in_specs=[
            pl.BlockSpec((_B, _CH3, _DIN), lambda k: (0, k, 0)),
            pl.BlockSpec((_B, 1, 1), lambda k: (0, 0, 0)),
            pl.BlockSpec((1, _DIN), lambda k: (0, 0)),
            pl.BlockSpec((1, _DIN), lambda k: (0, 0)),
            pl.BlockSpec((_DIN, _E), lambda k: (0, 0)),
            pl.BlockSpec((1, _E), lambda k: (0, 0)),
            pl.BlockSpec((_E, _E), lambda k: (0, 0)),
            pl.BlockSpec((1, _E), lambda k: (0, 0)),
            pl.BlockSpec((_B, 1, _G), lambda k: (0, 0, 0)),
            pl.BlockSpec((_E + _G, _C), lambda k: (0, 0)),
            pl.BlockSpec((1, _C), lambda k: (0, 0)),
        ],
        out_specs=[
            pl.BlockSpec((_B, _CH3, _C), lambda k: (0, k, 0)),
            pl.BlockSpec((_B, _CH3), lambda k: (0, k)),
            pl.BlockSpec((_B, _CH3), lambda k: (0, k)),
            pl.BlockSpec((_B, _CH3, _E + _G), lambda k: (0, k, 0)),
        ],
        out_shape=[
            jax.ShapeDtypeStruct((_B, _S, _C), f32),
            jax.ShapeDtypeStruct((_B, _S), jnp.int32),
            jax.ShapeDtypeStruct((_B, _S), f32),
            jax.ShapeDtypeStruct((_B, _S, _E + _G), f32),
        ],
    )(x, len3, mean_row, sv_row, W1, b1r, W2, b2r, gfv3, Wseg, bsegr)

    return (logits, labels[:, :, None], scores[:, :, None], cat)


# speedup vs baseline: 1.0451x; 1.0451x over previous
import jax
import jax.numpy as jnp
from jax.experimental import pallas as pl

_B, _S, _DIN, _E, _G, _C = 16, 4096, 8, 64, 128, 8
_R = _B * _DIN
_CH2 = 512
_CH3 = 256
_PAD = 0.0


def _stats_kernel(x2_ref, lrep_ref, mean_ref, sv_ref):
    f32 = jnp.float32
    x2 = x2_ref[...]
    lrep = lrep_ref[...]
    il = jax.lax.broadcasted_iota(jnp.int32, (_R, _S), 1)
    mf = (il < lrep).astype(f32)
    cnt = jnp.maximum(jnp.sum(mf) * (1.0 / _DIN), 1.0)

    rd = jax.lax.broadcasted_iota(jnp.int32, (_DIN, _R), 1)
    dd = jax.lax.broadcasted_iota(jnp.int32, (_DIN, _R), 0)
    m8 = (jax.lax.rem(rd, _DIN) == dd).astype(f32)
    rr = jax.lax.broadcasted_iota(jnp.int32, (_R, _DIN), 0)
    dc = jax.lax.broadcasted_iota(jnp.int32, (_R, _DIN), 1)
    p8 = (jax.lax.rem(rr, _DIN) == dc).astype(f32)

    hi = jax.lax.Precision.HIGHEST
    sum_rows = jnp.sum(x2 * mf, axis=1, keepdims=True)
    dn = (((1,), (0,)), ((), ()))
    mean8 = jax.lax.dot_general(m8, sum_rows, dn, precision=hi) / cnt
    mean_r = jax.lax.dot_general(p8, mean8, dn, precision=hi)
    xc = x2 - mean_r
    sq_rows = jnp.sum((xc * xc) * mf, axis=1, keepdims=True)
    var8 = jax.lax.dot_general(m8, sq_rows, dn, precision=hi) / cnt
    sv8 = jnp.sqrt(var8 + 1e-6)

    i8r = jax.lax.broadcasted_iota(jnp.int32, (_DIN, _DIN), 0)
    i8c = jax.lax.broadcasted_iota(jnp.int32, (_DIN, _DIN), 1)
    eye8 = (i8r == i8c).astype(f32)
    dt = (((0,), (0,)), ((), ()))
    mean_ref[...] = jax.lax.dot_general(mean8, eye8, dt, precision=hi)
    sv_ref[...] = jax.lax.dot_general(sv8, eye8, dt, precision=hi)


def _mlp_h(xs, mean_row, sv_row, w1, b1r, w2, b2r):
    xn = (xs - mean_row) / sv_row
    h = jnp.maximum(xn @ w1 + b1r, 0.0)
    return jnp.maximum(h @ w2 + b2r, 0.0)


def _rows_from_packed(x2):
    return jnp.concatenate(
        [jnp.transpose(x2[_DIN * b:_DIN * (b + 1), :]) for b in range(_B)],
        axis=0)


def _gfv_kernel(x2_ref, len3_ref, mean_ref, sv_ref, w1_ref, b1r_ref,
                w2_ref, b2r_ref, wg_ref, bgr_ref, gfv_ref):
    k = pl.program_id(0)
    base = k * _CH2
    len3 = len3_ref[...]

    h = _mlp_h(_rows_from_packed(x2_ref[...]), mean_ref[...], sv_ref[...],
               w1_ref[...], b1r_ref[...], w2_ref[...], b2r_ref[...])
    g = jnp.maximum(h @ wg_ref[...] + bgr_ref[...], 0.0)

    ig = jax.lax.broadcasted_iota(jnp.int32, (_B, _CH2, _G), 1) + base
    maskg = ig < jnp.broadcast_to(len3, (_B, _CH2, _G))
    g3 = jnp.where(maskg, g.reshape(_B, _CH2, _G), -jnp.inf)
    part = jnp.max(g3, axis=1)

    @pl.when(k == 0)
    def _():
        gfv_ref[...] = jnp.full((_B, _G), -jnp.inf, jnp.float32)

    gfv_ref[...] = jnp.maximum(gfv_ref[...], part)


def _out_kernel(x2_ref, len3_ref, mean_ref, sv_ref, w1_ref, b1r_ref,
                w2_ref, b2r_ref, gfv3_ref, wseg_ref, bsegr_ref,
                logits_ref, labels_ref, scores_ref, cat_ref):
    k = pl.program_id(0)
    base = k * _CH3
    len3 = len3_ref[...]

    n = _B * _CH3
    h = _mlp_h(_rows_from_packed(x2_ref[...]), mean_ref[...], sv_ref[...],
               w1_ref[...], b1r_ref[...], w2_ref[...], b2r_ref[...])

    ie = jax.lax.broadcasted_iota(jnp.int32, (_B, _CH3, _E), 1) + base
    maske = ie < jnp.broadcast_to(len3, (_B, _CH3, _E))
    h3 = jnp.where(maske, h.reshape(_B, _CH3, _E), _PAD)

    ig = jax.lax.broadcasted_iota(jnp.int32, (_B, _CH3, _G), 1) + base
    maskg = ig < jnp.broadcast_to(len3, (_B, _CH3, _G))
    gfv3 = jnp.broadcast_to(gfv3_ref[...], (_B, _CH3, _G))
    gfv3 = jnp.where(maskg, gfv3, _PAD)

    cat3 = jnp.concatenate([h3, gfv3], axis=2)
    cat_ref[...] = cat3

    logits = cat3.reshape(n, _E + _G) @ wseg_ref[...] + bsegr_ref[...]
    ic = jax.lax.broadcasted_iota(jnp.int32, (_B, _CH3, _C), 1) + base
    maskc = ic < jnp.broadcast_to(len3, (_B, _CH3, _C))
    logits3 = jnp.where(maskc, logits.reshape(_B, _CH3, _C), _PAD)
    logits_ref[...] = logits3

    m = jnp.max(logits3, axis=2, keepdims=True)
    ssum = jnp.sum(jnp.exp(logits3 - m), axis=2)
    scores = 1.0 / ssum
    scores_ref[...] = scores

    cidx = jax.lax.broadcasted_iota(jnp.int32, (_B, _CH3, _C), 2)
    cand = jnp.where(logits3 == m, cidx, _C)
    labels = jnp.min(cand, axis=2)
    labels_ref[...] = jnp.where(jnp.isnan(scores), -1, labels)


def kernel(x, lengths, W1, b1, W2, b2, Wg, bg, Wseg, bseg):
    f32 = jnp.float32
    x2 = x.transpose(0, 2, 1).reshape(_R, _S)
    lrep = jnp.repeat(lengths.astype(jnp.int32), _DIN).reshape(_R, 1)
    len3 = lengths.astype(jnp.int32).reshape(_B, 1, 1)
    b1r = b1.reshape(1, _E)
    b2r = b2.reshape(1, _E)
    bgr = bg.reshape(1, _G)
    bsegr = bseg.reshape(1, _C)

    mean_row, sv_row = pl.pallas_call(
        _stats_kernel,
        out_shape=[
            jax.ShapeDtypeStruct((1, _DIN), f32),
            jax.ShapeDtypeStruct((1, _DIN), f32),
        ],
    )(x2, lrep)

    n2 = _S // _CH2
    gfv = pl.pallas_call(
        _gfv_kernel,
        grid=(n2,),
        in_specs=[
            pl.BlockSpec((_R, _CH2), lambda k: (0, k)),
            pl.BlockSpec((_B, 1, 1), lambda k: (0, 0, 0)),
            pl.BlockSpec((1, _DIN), lambda k: (0, 0)),
            pl.BlockSpec((1, _DIN), lambda k: (0, 0)),
            pl.BlockSpec((_DIN, _E), lambda k: (0, 0)),
            pl.BlockSpec((1, _E), lambda k: (0, 0)),
            pl.BlockSpec((_E, _E), lambda k: (0, 0)),
            pl.BlockSpec((1, _E), lambda k: (0, 0)),
            pl.BlockSpec((_E, _G), lambda k: (0, 0)),
            pl.BlockSpec((1, _G), lambda k: (0, 0)),
        ],
        out_specs=pl.BlockSpec((_B, _G), lambda k: (0, 0)),
        out_shape=jax.ShapeDtypeStruct((_B, _G), f32),
    )(x2, len3, mean_row, sv_row, W1, b1r, W2, b2r, Wg, bgr)

    gfv3 = gfv.reshape(_B, 1, _G)

    n3 = _S // _CH3
    logits, labels, scores, cat = pl.pallas_call(
        _out_kernel,
        grid=(n3,),
        in_specs=[
            pl.BlockSpec((_R, _CH3), lambda k: (0, k)),
            pl.BlockSpec((_B, 1, 1), lambda k: (0, 0, 0)),
            pl.BlockSpec((1, _DIN), lambda k: (0, 0)),
            pl.BlockSpec((1, _DIN), lambda k: (0, 0)),
            pl.BlockSpec((_DIN, _E), lambda k: (0, 0)),
            pl.BlockSpec((1, _E), lambda k: (0, 0)),
            pl.BlockSpec((_E, _E), lambda k: (0, 0)),
            pl.BlockSpec((1, _E), lambda k: (0, 0)),
            pl.BlockSpec((_B, 1, _G), lambda k: (0, 0, 0)),
            pl.BlockSpec((_E + _G, _C), lambda k: (0, 0)),
            pl.BlockSpec((1, _C), lambda k: (0, 0)),
        ],
        out_specs=[
            pl.BlockSpec((_B, _CH3, _C), lambda k: (0, k, 0)),
            pl.BlockSpec((_B, _CH3), lambda k: (0, k)),
            pl.BlockSpec((_B, _CH3), lambda k: (0, k)),
            pl.BlockSpec((_B, _CH3, _E + _G), lambda k: (0, k, 0)),
        ],
        out_shape=[
            jax.ShapeDtypeStruct((_B, _S, _C), f32),
            jax.ShapeDtypeStruct((_B, _S), jnp.int32),
            jax.ShapeDtypeStruct((_B, _S), f32),
            jax.ShapeDtypeStruct((_B, _S, _E + _G), f32),
        ],
    )(x2, len3, mean_row, sv_row, W1, b1r, W2, b2r, gfv3, Wseg, bsegr)

    return (logits, labels[:, :, None], scores[:, :, None], cat)
